# Initial kernel scaffold; baseline (speedup 1.0000x reference)
import jax
import jax.numpy as jnp
from jax.experimental import pallas as pl


def kernel(scores, bbox_deltas, im_info):
    raise NotImplementedError("write your pallas kernel here")


# speedup vs baseline: 74.4547x; 74.4547x over previous
import functools

import jax
import jax.numpy as jnp
import numpy as np
from jax import lax
from jax.experimental import pallas as pl
from jax.experimental.pallas import tpu as pltpu
from jax.experimental.pallas import tpu_sc as plsc

PRE_NMS_TOPN = 6000
POST_NMS_TOPN = 1000
NMS_THRESH = 0.7
IMG_SIZE = 512
BATCH = 4

CAND_PAD = 6016
KEEP_PAD = 1024
LANES = 16


def _gen_anchors():
    pyramid_levels = [3, 4, 5, 6, 7]
    ratios = np.array([0.5, 1.0, 2.0])
    scales = np.array([2 ** 0, 2 ** (1.0 / 3.0), 2 ** (2.0 / 3.0)])
    image_shape = np.array([IMG_SIZE, IMG_SIZE])
    all_anchors = np.zeros((0, 4), dtype=np.float64)
    for p in pyramid_levels:
        base_size = 2 ** (p + 2)
        stride = 2 ** p
        num_anchors = len(ratios) * len(scales)
        anchors = np.zeros((num_anchors, 4))
        anchors[:, 2:] = base_size * np.tile(scales, (2, len(ratios))).T
        areas = anchors[:, 2] * anchors[:, 3]
        anchors[:, 2] = np.sqrt(areas / np.repeat(ratios, len(scales)))
        anchors[:, 3] = anchors[:, 2] * np.repeat(ratios, len(scales))
        anchors[:, 0::2] -= np.tile(anchors[:, 2] * 0.5, (2, 1)).T
        anchors[:, 1::2] -= np.tile(anchors[:, 3] * 0.5, (2, 1)).T
        shape = (image_shape + stride - 1) // stride
        shift_x = (np.arange(0, shape[1]) + 0.5) * stride
        shift_y = (np.arange(0, shape[0]) + 0.5) * stride
        sx, sy = np.meshgrid(shift_x, shift_y)
        shifts = np.vstack((sx.ravel(), sy.ravel(), sx.ravel(), sy.ravel())).transpose()
        A = anchors.shape[0]
        K = shifts.shape[0]
        shifted = (anchors.reshape((1, A, 4)) + shifts.reshape((1, K, 4)).transpose((1, 0, 2))).reshape((K * A, 4))
        all_anchors = np.append(all_anchors, shifted, axis=0)
    return all_anchors.astype(np.float32)


_ANCHORS = _gen_anchors()


def _nms_body(a0, a1, a2, a3, d0, d1, e2, e3, out_hbm,
              sa0, sa1, sa2, sa3, sd0, sd1, se2, se3,
              vx1, vy1, vx2, vy2, var,
              kx1, ky1, kx2, ky2, kar, c0):
    cid = lax.axis_index("c")
    sid = lax.axis_index("s")
    wid = sid * 2 + cid
    i = lax.rem(wid, BATCH)

    for src, dst in ((a0, sa0), (a1, sa1), (a2, sa2), (a3, sa3),
                     (d0, sd0), (d1, sd1), (e2, se2), (e3, se3)):
        pltpu.sync_copy(src.at[pl.ds(i * CAND_PAD, CAND_PAD)], dst)

    if True:

        def tbody(c, _):
            sl = pl.ds(c * LANES, LANES)
            w = sa2[sl] - sa0[sl]
            h = sa3[sl] - sa1[sl]
            cx = sa0[sl] + 0.5 * w
            cy = sa1[sl] + 0.5 * h
            pcx = cx + (sd0[sl] * 0.1) * w
            pcy = cy + (sd1[sl] * 0.1) * h
            pw = se2[sl] * w
            ph = se3[sl] * h
            x1 = jnp.minimum(jnp.maximum(pcx - 0.5 * pw, 0.0), float(IMG_SIZE))
            y1 = jnp.minimum(jnp.maximum(pcy - 0.5 * ph, 0.0), float(IMG_SIZE))
            x2 = jnp.minimum(jnp.maximum(pcx + 0.5 * pw, 0.0), float(IMG_SIZE))
            y2 = jnp.minimum(jnp.maximum(pcy + 0.5 * ph, 0.0), float(IMG_SIZE))
            vx1[sl] = x1
            vy1[sl] = y1
            vx2[sl] = x2
            vy2[sl] = y2
            var[sl] = (x2 - x1) * (y2 - y1)
            return 0

        lax.fori_loop(0, CAND_PAD // LANES, tbody, 0)

        ifl = lax.convert_element_type(i, jnp.float32)

        def zbody(c, _):
            sl = pl.ds(c * LANES, LANES)
            z = jnp.zeros((LANES,), jnp.float32)
            kx1[sl] = z
            ky1[sl] = z
            kx2[sl] = z
            ky2[sl] = z
            c0[sl] = jnp.full((LANES,), ifl, jnp.float32)
            return 0

        lax.fori_loop(0, KEEP_PAD // LANES, zbody, 0)

        lane = lax.iota(jnp.int32, LANES)
        wmask = lane == 0

        def body(j, cnt):
            jv = jnp.full((LANES,), j, jnp.int32)
            bx1 = plsc.load_gather(vx1, [jv])
            by1 = plsc.load_gather(vy1, [jv])
            bx2 = plsc.load_gather(vx2, [jv])
            by2 = plsc.load_gather(vy2, [jv])
            bar = plsc.load_gather(var, [jv])

            def ibody(c, sup):
                sl = pl.ds(c * LANES, LANES)
                xx1 = jnp.maximum(kx1[sl], bx1)
                yy1 = jnp.maximum(ky1[sl], by1)
                xx2 = jnp.minimum(kx2[sl], bx2)
                yy2 = jnp.minimum(ky2[sl], by2)
                w = jnp.maximum(xx2 - xx1, 0.0)
                h = jnp.maximum(yy2 - yy1, 0.0)
                inter = w * h
                iou = inter / (kar[sl] + bar - inter + 1e-8)
                hit = jnp.logical_and(iou > NMS_THRESH, (c * LANES + lane) < cnt)
                return jnp.logical_or(sup, jnp.any(hit))

            active = cnt < POST_NMS_TOPN
            nch = jnp.where(active, (cnt + LANES - 1) // LANES, jnp.int32(0))
            sup = lax.fori_loop(0, nch, ibody, jnp.bool_(False))

            take = jnp.logical_and(jnp.logical_not(sup), active)
            smask = jnp.logical_and(wmask, jnp.full((LANES,), take))
            cv = jnp.full((LANES,), cnt, jnp.int32)
            plsc.store_scatter(kx1, [cv], bx1, mask=smask)
            plsc.store_scatter(ky1, [cv], by1, mask=smask)
            plsc.store_scatter(kx2, [cv], bx2, mask=smask)
            plsc.store_scatter(ky2, [cv], by2, mask=smask)
            plsc.store_scatter(kar, [cv], bar, mask=smask)

            return cnt + jnp.where(take, jnp.int32(1), jnp.int32(0))

        def bbody(b, cnt):
            nin = jnp.where(cnt < POST_NMS_TOPN, jnp.int32(LANES), jnp.int32(0))

            def cbody(t, c):
                return body(b * LANES + t, c)

            return lax.fori_loop(0, nin, cbody, cnt)

        lax.fori_loop(0, PRE_NMS_TOPN // LANES, bbody, jnp.int32(0))

        base = wid * 5 * KEEP_PAD
        pltpu.sync_copy(c0, out_hbm.at[pl.ds(base, KEEP_PAD)])
        pltpu.sync_copy(kx1, out_hbm.at[pl.ds(base + KEEP_PAD, KEEP_PAD)])
        pltpu.sync_copy(ky1, out_hbm.at[pl.ds(base + 2 * KEEP_PAD, KEEP_PAD)])
        pltpu.sync_copy(kx2, out_hbm.at[pl.ds(base + 3 * KEEP_PAD, KEEP_PAD)])
        pltpu.sync_copy(ky2, out_hbm.at[pl.ds(base + 4 * KEEP_PAD, KEEP_PAD)])


_nms_sc = functools.partial(
    pl.kernel,
    out_type=jax.ShapeDtypeStruct((32 * 5 * KEEP_PAD,), jnp.float32),
    mesh=plsc.VectorSubcoreMesh(core_axis_name="c", subcore_axis_name="s"),
    compiler_params=pltpu.CompilerParams(needs_layout_passes=False),
    scratch_types=(
        [pltpu.VMEM((CAND_PAD,), jnp.float32) for _ in range(8)]
        + [pltpu.VMEM((CAND_PAD,), jnp.float32) for _ in range(5)]
        + [pltpu.VMEM((KEEP_PAD,), jnp.float32) for _ in range(6)]
    ),
)(_nms_body)


def kernel(scores, bbox_deltas, im_info):
    del im_info
    sc = scores[:, :, 0]
    _, idx = lax.top_k(sc, PRE_NMS_TOPN)

    anchors = jnp.asarray(_ANCHORS)
    anc = anchors[idx]
    dg = jnp.take_along_axis(bbox_deltas, idx[:, :, None], axis=1)

    pad = ((0, 0), (0, CAND_PAD - PRE_NMS_TOPN))

    def planar(x):
        return jnp.pad(x, pad).reshape(-1)

    a0 = planar(anc[:, :, 0])
    a1 = planar(anc[:, :, 1])
    a2 = planar(anc[:, :, 2])
    a3 = planar(anc[:, :, 3])
    d0 = planar(dg[:, :, 0])
    d1 = planar(dg[:, :, 1])
    e2 = planar(jnp.exp(dg[:, :, 2] * 0.2))
    e3 = planar(jnp.exp(dg[:, :, 3] * 0.2))

    out = _nms_sc(a0, a1, a2, a3, d0, d1, e2, e3)
    out = out.reshape(32, 5, KEEP_PAD)[:BATCH]
    return jnp.transpose(out, (0, 2, 1))[:, :POST_NMS_TOPN, :]


# speedup vs baseline: 78.8213x; 1.0586x over previous
import functools

import jax
import jax.numpy as jnp
import numpy as np
from jax import lax
from jax.experimental import pallas as pl
from jax.experimental.pallas import tpu as pltpu
from jax.experimental.pallas import tpu_sc as plsc

PRE_NMS_TOPN = 6000
POST_NMS_TOPN = 1000
NMS_THRESH = 0.7
IMG_SIZE = 512
BATCH = 4

CAND_PAD = 6016
KEEP_PAD = 1024
LANES = 16


def _gen_anchors():
    pyramid_levels = [3, 4, 5, 6, 7]
    ratios = np.array([0.5, 1.0, 2.0])
    scales = np.array([2 ** 0, 2 ** (1.0 / 3.0), 2 ** (2.0 / 3.0)])
    image_shape = np.array([IMG_SIZE, IMG_SIZE])
    all_anchors = np.zeros((0, 4), dtype=np.float64)
    for p in pyramid_levels:
        base_size = 2 ** (p + 2)
        stride = 2 ** p
        num_anchors = len(ratios) * len(scales)
        anchors = np.zeros((num_anchors, 4))
        anchors[:, 2:] = base_size * np.tile(scales, (2, len(ratios))).T
        areas = anchors[:, 2] * anchors[:, 3]
        anchors[:, 2] = np.sqrt(areas / np.repeat(ratios, len(scales)))
        anchors[:, 3] = anchors[:, 2] * np.repeat(ratios, len(scales))
        anchors[:, 0::2] -= np.tile(anchors[:, 2] * 0.5, (2, 1)).T
        anchors[:, 1::2] -= np.tile(anchors[:, 3] * 0.5, (2, 1)).T
        shape = (image_shape + stride - 1) // stride
        shift_x = (np.arange(0, shape[1]) + 0.5) * stride
        shift_y = (np.arange(0, shape[0]) + 0.5) * stride
        sx, sy = np.meshgrid(shift_x, shift_y)
        shifts = np.vstack((sx.ravel(), sy.ravel(), sx.ravel(), sy.ravel())).transpose()
        A = anchors.shape[0]
        K = shifts.shape[0]
        shifted = (anchors.reshape((1, A, 4)) + shifts.reshape((1, K, 4)).transpose((1, 0, 2))).reshape((K * A, 4))
        all_anchors = np.append(all_anchors, shifted, axis=0)
    return all_anchors.astype(np.float32)


_ANCHORS = _gen_anchors()


def _nms_body(a0, a1, a2, a3, d0, d1, e2, e3, out_hbm,
              sa0, sa1, sa2, sa3, sd0, sd1, se2, se3,
              vx1, vy1, vx2, vy2, var,
              kx1, ky1, kx2, ky2, kar, c0):
    cid = lax.axis_index("c")
    sid = lax.axis_index("s")
    wid = sid * 2 + cid
    i = lax.rem(wid, BATCH)

    for src, dst in ((a0, sa0), (a1, sa1), (a2, sa2), (a3, sa3),
                     (d0, sd0), (d1, sd1), (e2, se2), (e3, se3)):
        pltpu.sync_copy(src.at[pl.ds(i * CAND_PAD, CAND_PAD)], dst)

    if True:

        def tbody(c, _):
            sl = pl.ds(c * LANES, LANES)
            w = sa2[sl] - sa0[sl]
            h = sa3[sl] - sa1[sl]
            cx = sa0[sl] + 0.5 * w
            cy = sa1[sl] + 0.5 * h
            pcx = cx + (sd0[sl] * 0.1) * w
            pcy = cy + (sd1[sl] * 0.1) * h
            pw = se2[sl] * w
            ph = se3[sl] * h
            x1 = jnp.minimum(jnp.maximum(pcx - 0.5 * pw, 0.0), float(IMG_SIZE))
            y1 = jnp.minimum(jnp.maximum(pcy - 0.5 * ph, 0.0), float(IMG_SIZE))
            x2 = jnp.minimum(jnp.maximum(pcx + 0.5 * pw, 0.0), float(IMG_SIZE))
            y2 = jnp.minimum(jnp.maximum(pcy + 0.5 * ph, 0.0), float(IMG_SIZE))
            vx1[sl] = x1
            vy1[sl] = y1
            vx2[sl] = x2
            vy2[sl] = y2
            var[sl] = (x2 - x1) * (y2 - y1)
            return 0

        lax.fori_loop(0, CAND_PAD // LANES, tbody, 0)

        ifl = lax.convert_element_type(i, jnp.float32)

        def zbody(c, _):
            sl = pl.ds(c * LANES, LANES)
            far = jnp.full((LANES,), 1e9, jnp.float32)
            kx1[sl] = far
            ky1[sl] = far
            kx2[sl] = far
            ky2[sl] = far
            kar[sl] = jnp.zeros((LANES,), jnp.float32)
            c0[sl] = jnp.full((LANES,), ifl, jnp.float32)
            return 0

        lax.fori_loop(0, KEEP_PAD // LANES, zbody, 0)

        lane = lax.iota(jnp.int32, LANES)
        wmask = lane == 0

        def body(j, cnt):
            jv = jnp.full((LANES,), j, jnp.int32)
            bx1 = plsc.load_gather(vx1, [jv])
            by1 = plsc.load_gather(vy1, [jv])
            bx2 = plsc.load_gather(vx2, [jv])
            by2 = plsc.load_gather(vy2, [jv])
            bar = plsc.load_gather(var, [jv])

            def _hits(off):
                sl = pl.ds(off, LANES)
                xx1 = jnp.maximum(kx1[sl], bx1)
                yy1 = jnp.maximum(ky1[sl], by1)
                xx2 = jnp.minimum(kx2[sl], bx2)
                yy2 = jnp.minimum(ky2[sl], by2)
                w = jnp.maximum(xx2 - xx1, 0.0)
                h = jnp.maximum(yy2 - yy1, 0.0)
                inter = w * h
                iou = inter / (kar[sl] + bar - inter + 1e-8)
                return iou > NMS_THRESH

            def ibody(c, supv):
                off = c * (2 * LANES)
                return supv | _hits(off) | _hits(off + LANES)

            active = cnt < POST_NMS_TOPN
            nch = jnp.where(active, (cnt + 2 * LANES - 1) // (2 * LANES),
                            jnp.int32(0))
            supv = lax.fori_loop(0, nch, ibody,
                                 jnp.zeros((LANES,), jnp.bool_))
            sup = jnp.any(supv)

            take = jnp.logical_and(jnp.logical_not(sup), active)
            smask = jnp.logical_and(wmask, jnp.full((LANES,), take))
            cv = jnp.full((LANES,), cnt, jnp.int32)
            plsc.store_scatter(kx1, [cv], bx1, mask=smask)
            plsc.store_scatter(ky1, [cv], by1, mask=smask)
            plsc.store_scatter(kx2, [cv], bx2, mask=smask)
            plsc.store_scatter(ky2, [cv], by2, mask=smask)
            plsc.store_scatter(kar, [cv], bar, mask=smask)

            return cnt + jnp.where(take, jnp.int32(1), jnp.int32(0))

        def bbody(b, cnt):
            nin = jnp.where(cnt < POST_NMS_TOPN, jnp.int32(LANES), jnp.int32(0))

            def cbody(t, c):
                return body(b * LANES + t, c)

            return lax.fori_loop(0, nin, cbody, cnt)

        cnt_f = lax.fori_loop(0, PRE_NMS_TOPN // LANES, bbody, jnp.int32(0))

        def fbody(c, _):
            sl = pl.ds(c * LANES, LANES)
            m = (c * LANES + lane) < cnt_f
            z = jnp.zeros((LANES,), jnp.float32)
            kx1[sl] = jnp.where(m, kx1[sl], z)
            ky1[sl] = jnp.where(m, ky1[sl], z)
            kx2[sl] = jnp.where(m, kx2[sl], z)
            ky2[sl] = jnp.where(m, ky2[sl], z)
            return 0

        lax.fori_loop(0, KEEP_PAD // LANES, fbody, 0)

        base = wid * 5 * KEEP_PAD
        pltpu.sync_copy(c0, out_hbm.at[pl.ds(base, KEEP_PAD)])
        pltpu.sync_copy(kx1, out_hbm.at[pl.ds(base + KEEP_PAD, KEEP_PAD)])
        pltpu.sync_copy(ky1, out_hbm.at[pl.ds(base + 2 * KEEP_PAD, KEEP_PAD)])
        pltpu.sync_copy(kx2, out_hbm.at[pl.ds(base + 3 * KEEP_PAD, KEEP_PAD)])
        pltpu.sync_copy(ky2, out_hbm.at[pl.ds(base + 4 * KEEP_PAD, KEEP_PAD)])


_nms_sc = functools.partial(
    pl.kernel,
    out_type=jax.ShapeDtypeStruct((32 * 5 * KEEP_PAD,), jnp.float32),
    mesh=plsc.VectorSubcoreMesh(core_axis_name="c", subcore_axis_name="s"),
    compiler_params=pltpu.CompilerParams(needs_layout_passes=False),
    scratch_types=(
        [pltpu.VMEM((CAND_PAD,), jnp.float32) for _ in range(8)]
        + [pltpu.VMEM((CAND_PAD,), jnp.float32) for _ in range(5)]
        + [pltpu.VMEM((KEEP_PAD,), jnp.float32) for _ in range(6)]
    ),
)(_nms_body)


def kernel(scores, bbox_deltas, im_info):
    del im_info
    sc = scores[:, :, 0]
    _, idx = lax.top_k(sc, PRE_NMS_TOPN)

    anchors = jnp.asarray(_ANCHORS)
    anc = anchors[idx]
    dg = jnp.take_along_axis(bbox_deltas, idx[:, :, None], axis=1)

    pad = ((0, 0), (0, CAND_PAD - PRE_NMS_TOPN))

    def planar(x):
        return jnp.pad(x, pad).reshape(-1)

    a0 = planar(anc[:, :, 0])
    a1 = planar(anc[:, :, 1])
    a2 = planar(anc[:, :, 2])
    a3 = planar(anc[:, :, 3])
    d0 = planar(dg[:, :, 0])
    d1 = planar(dg[:, :, 1])
    e2 = planar(jnp.exp(dg[:, :, 2] * 0.2))
    e3 = planar(jnp.exp(dg[:, :, 3] * 0.2))

    out = _nms_sc(a0, a1, a2, a3, d0, d1, e2, e3)
    out = out.reshape(32, 5, KEEP_PAD)[:BATCH]
    return jnp.transpose(out, (0, 2, 1))[:, :POST_NMS_TOPN, :]


# speedup vs baseline: 105.4893x; 1.3383x over previous
import functools

import jax
import jax.numpy as jnp
import numpy as np
from jax import lax
from jax.experimental import pallas as pl
from jax.experimental.pallas import tpu as pltpu
from jax.experimental.pallas import tpu_sc as plsc

PRE_NMS_TOPN = 6000
POST_NMS_TOPN = 1000
NMS_THRESH = 0.7
IMG_SIZE = 512
BATCH = 4

CAND_PAD = 6016
KEEP_PAD = 1024
LANES = 16


def _gen_anchors():
    pyramid_levels = [3, 4, 5, 6, 7]
    ratios = np.array([0.5, 1.0, 2.0])
    scales = np.array([2 ** 0, 2 ** (1.0 / 3.0), 2 ** (2.0 / 3.0)])
    image_shape = np.array([IMG_SIZE, IMG_SIZE])
    all_anchors = np.zeros((0, 4), dtype=np.float64)
    for p in pyramid_levels:
        base_size = 2 ** (p + 2)
        stride = 2 ** p
        num_anchors = len(ratios) * len(scales)
        anchors = np.zeros((num_anchors, 4))
        anchors[:, 2:] = base_size * np.tile(scales, (2, len(ratios))).T
        areas = anchors[:, 2] * anchors[:, 3]
        anchors[:, 2] = np.sqrt(areas / np.repeat(ratios, len(scales)))
        anchors[:, 3] = anchors[:, 2] * np.repeat(ratios, len(scales))
        anchors[:, 0::2] -= np.tile(anchors[:, 2] * 0.5, (2, 1)).T
        anchors[:, 1::2] -= np.tile(anchors[:, 3] * 0.5, (2, 1)).T
        shape = (image_shape + stride - 1) // stride
        shift_x = (np.arange(0, shape[1]) + 0.5) * stride
        shift_y = (np.arange(0, shape[0]) + 0.5) * stride
        sx, sy = np.meshgrid(shift_x, shift_y)
        shifts = np.vstack((sx.ravel(), sy.ravel(), sx.ravel(), sy.ravel())).transpose()
        A = anchors.shape[0]
        K = shifts.shape[0]
        shifted = (anchors.reshape((1, A, 4)) + shifts.reshape((1, K, 4)).transpose((1, 0, 2))).reshape((K * A, 4))
        all_anchors = np.append(all_anchors, shifted, axis=0)
    return all_anchors.astype(np.float32)


_ANCHORS = _gen_anchors()


def _nms_body(a0, a1, a2, a3, d0, d1, e2, e3, out_hbm,
              sa0, sa1, sa2, sa3, sd0, sd1, se2, se3,
              vx1, vy1, vx2, vy2, var,
              kx1, ky1, kx2, ky2, kar, c0):
    cid = lax.axis_index("c")
    sid = lax.axis_index("s")
    wid = sid * 2 + cid
    i = lax.rem(wid, BATCH)

    for src, dst in ((a0, sa0), (a1, sa1), (a2, sa2), (a3, sa3),
                     (d0, sd0), (d1, sd1), (e2, se2), (e3, se3)):
        pltpu.sync_copy(src.at[pl.ds(i * CAND_PAD, CAND_PAD)], dst)

    if True:

        def tbody(c, _):
            sl = pl.ds(c * LANES, LANES)
            w = sa2[sl] - sa0[sl]
            h = sa3[sl] - sa1[sl]
            cx = sa0[sl] + 0.5 * w
            cy = sa1[sl] + 0.5 * h
            pcx = cx + (sd0[sl] * 0.1) * w
            pcy = cy + (sd1[sl] * 0.1) * h
            pw = se2[sl] * w
            ph = se3[sl] * h
            x1 = jnp.minimum(jnp.maximum(pcx - 0.5 * pw, 0.0), float(IMG_SIZE))
            y1 = jnp.minimum(jnp.maximum(pcy - 0.5 * ph, 0.0), float(IMG_SIZE))
            x2 = jnp.minimum(jnp.maximum(pcx + 0.5 * pw, 0.0), float(IMG_SIZE))
            y2 = jnp.minimum(jnp.maximum(pcy + 0.5 * ph, 0.0), float(IMG_SIZE))
            vx1[sl] = x1
            vy1[sl] = y1
            vx2[sl] = x2
            vy2[sl] = y2
            var[sl] = (x2 - x1) * (y2 - y1)
            return 0

        lax.fori_loop(0, CAND_PAD // LANES, tbody, 0)

        ifl = lax.convert_element_type(i, jnp.float32)

        def zbody(c, _):
            sl = pl.ds(c * LANES, LANES)
            far = jnp.full((LANES,), 1e9, jnp.float32)
            kx1[sl] = far
            ky1[sl] = far
            kx2[sl] = far
            ky2[sl] = far
            kar[sl] = jnp.zeros((LANES,), jnp.float32)
            c0[sl] = jnp.full((LANES,), ifl, jnp.float32)
            return 0

        lax.fori_loop(0, KEEP_PAD // LANES, zbody, 0)

        lane = lax.iota(jnp.int32, LANES)
        wmask = lane == 0

        def body(j, cnt):
            jv = jnp.full((LANES,), j, jnp.int32)
            bx1 = plsc.load_gather(vx1, [jv])
            by1 = plsc.load_gather(vy1, [jv])
            bx2 = plsc.load_gather(vx2, [jv])
            by2 = plsc.load_gather(vy2, [jv])
            bar = plsc.load_gather(var, [jv])

            def _hits(off):
                sl = pl.ds(off, LANES)
                xx1 = jnp.maximum(kx1[sl], bx1)
                yy1 = jnp.maximum(ky1[sl], by1)
                xx2 = jnp.minimum(kx2[sl], bx2)
                yy2 = jnp.minimum(ky2[sl], by2)
                w = jnp.maximum(xx2 - xx1, 0.0)
                h = jnp.maximum(yy2 - yy1, 0.0)
                inter = w * h
                iou = inter / (kar[sl] + bar - inter + 1e-8)
                return iou > NMS_THRESH

            def ibody(c, supv):
                off = c * (2 * LANES)
                return supv | _hits(off) | _hits(off + LANES)

            active = cnt < POST_NMS_TOPN
            nch = jnp.where(active, (cnt + 2 * LANES - 1) // (2 * LANES),
                            jnp.int32(0))
            supv = lax.fori_loop(0, nch, ibody,
                                 jnp.zeros((LANES,), jnp.bool_))
            sup = jnp.any(supv)

            take = jnp.logical_and(jnp.logical_not(sup), active)
            smask = jnp.logical_and(wmask, jnp.full((LANES,), take))
            cv = jnp.full((LANES,), cnt, jnp.int32)
            plsc.store_scatter(kx1, [cv], bx1, mask=smask)
            plsc.store_scatter(ky1, [cv], by1, mask=smask)
            plsc.store_scatter(kx2, [cv], bx2, mask=smask)
            plsc.store_scatter(ky2, [cv], by2, mask=smask)
            plsc.store_scatter(kar, [cv], bar, mask=smask)

            return cnt + jnp.where(take, jnp.int32(1), jnp.int32(0))

        def bbody(b, cnt):
            nin = jnp.where(cnt < POST_NMS_TOPN, jnp.int32(LANES), jnp.int32(0))

            def cbody(t, c):
                return body(b * LANES + t, c)

            return lax.fori_loop(0, nin, cbody, cnt)

        cnt_f = lax.fori_loop(0, PRE_NMS_TOPN // LANES, bbody, jnp.int32(0))

        def fbody(c, _):
            sl = pl.ds(c * LANES, LANES)
            m = (c * LANES + lane) < cnt_f
            z = jnp.zeros((LANES,), jnp.float32)
            kx1[sl] = jnp.where(m, kx1[sl], z)
            ky1[sl] = jnp.where(m, ky1[sl], z)
            kx2[sl] = jnp.where(m, kx2[sl], z)
            ky2[sl] = jnp.where(m, ky2[sl], z)
            return 0

        lax.fori_loop(0, KEEP_PAD // LANES, fbody, 0)

        base = wid * 5 * KEEP_PAD
        pltpu.sync_copy(c0, out_hbm.at[pl.ds(base, KEEP_PAD)])
        pltpu.sync_copy(kx1, out_hbm.at[pl.ds(base + KEEP_PAD, KEEP_PAD)])
        pltpu.sync_copy(ky1, out_hbm.at[pl.ds(base + 2 * KEEP_PAD, KEEP_PAD)])
        pltpu.sync_copy(kx2, out_hbm.at[pl.ds(base + 3 * KEEP_PAD, KEEP_PAD)])
        pltpu.sync_copy(ky2, out_hbm.at[pl.ds(base + 4 * KEEP_PAD, KEEP_PAD)])


_nms_sc = functools.partial(
    pl.kernel,
    out_type=jax.ShapeDtypeStruct((32 * 5 * KEEP_PAD,), jnp.float32),
    mesh=plsc.VectorSubcoreMesh(core_axis_name="c", subcore_axis_name="s"),
    compiler_params=pltpu.CompilerParams(needs_layout_passes=False),
    scratch_types=(
        [pltpu.VMEM((CAND_PAD,), jnp.float32) for _ in range(8)]
        + [pltpu.VMEM((CAND_PAD,), jnp.float32) for _ in range(5)]
        + [pltpu.VMEM((KEEP_PAD,), jnp.float32) for _ in range(6)]
    ),
)(_nms_body)


NUM_ANCH = _ANCHORS.shape[0]
ANCH_PAD = 49152
N_SHARD = 16
SHARD = ANCH_PAD // N_SHARD
KSHARD = 512


def _topk_sharded(sc):
    scp = jnp.pad(sc, ((0, 0), (0, ANCH_PAD - NUM_ANCH)),
                  constant_values=-1.0)
    sh = scp.reshape(BATCH, N_SHARD, SHARD)
    v1, i1 = lax.top_k(sh, KSHARD)
    gidx = i1 + (jnp.arange(N_SHARD, dtype=jnp.int32) * SHARD)[None, :, None]
    vm = v1.reshape(BATCH, N_SHARD * KSHARD)
    im = gidx.reshape(BATCH, N_SHARD * KSHARD)
    v2, i2 = lax.top_k(vm, PRE_NMS_TOPN)
    idx = jnp.take_along_axis(im, i2, axis=1)
    tau = v2[:, PRE_NMS_TOPN - 1]
    safe = jnp.all(v1[:, :, KSHARD - 1] < tau[:, None])
    return lax.cond(safe, lambda: idx,
                    lambda: lax.top_k(sc, PRE_NMS_TOPN)[1])


def kernel(scores, bbox_deltas, im_info):
    del im_info
    sc = scores[:, :, 0]
    idx = _topk_sharded(sc)

    anchors = jnp.asarray(_ANCHORS)
    anc = anchors[idx]
    dg = jnp.take_along_axis(bbox_deltas, idx[:, :, None], axis=1)

    pad = ((0, 0), (0, CAND_PAD - PRE_NMS_TOPN))

    def planar(x):
        return jnp.pad(x, pad).reshape(-1)

    a0 = planar(anc[:, :, 0])
    a1 = planar(anc[:, :, 1])
    a2 = planar(anc[:, :, 2])
    a3 = planar(anc[:, :, 3])
    d0 = planar(dg[:, :, 0])
    d1 = planar(dg[:, :, 1])
    e2 = planar(jnp.exp(dg[:, :, 2] * 0.2))
    e3 = planar(jnp.exp(dg[:, :, 3] * 0.2))

    out = _nms_sc(a0, a1, a2, a3, d0, d1, e2, e3)
    out = out.reshape(32, 5, KEEP_PAD)[:BATCH]
    return jnp.transpose(out, (0, 2, 1))[:, :POST_NMS_TOPN, :]


# speedup vs baseline: 119.7862x; 1.1355x over previous
import functools

import jax
import jax.numpy as jnp
import numpy as np
from jax import lax
from jax.experimental import pallas as pl
from jax.experimental.pallas import tpu as pltpu
from jax.experimental.pallas import tpu_sc as plsc

PRE_NMS_TOPN = 6000
POST_NMS_TOPN = 1000
NMS_THRESH = 0.7
IMG_SIZE = 512
BATCH = 4

CAND_PAD = 6016
KEEP_PAD = 1024
LANES = 16


def _gen_anchors():
    pyramid_levels = [3, 4, 5, 6, 7]
    ratios = np.array([0.5, 1.0, 2.0])
    scales = np.array([2 ** 0, 2 ** (1.0 / 3.0), 2 ** (2.0 / 3.0)])
    image_shape = np.array([IMG_SIZE, IMG_SIZE])
    all_anchors = np.zeros((0, 4), dtype=np.float64)
    for p in pyramid_levels:
        base_size = 2 ** (p + 2)
        stride = 2 ** p
        num_anchors = len(ratios) * len(scales)
        anchors = np.zeros((num_anchors, 4))
        anchors[:, 2:] = base_size * np.tile(scales, (2, len(ratios))).T
        areas = anchors[:, 2] * anchors[:, 3]
        anchors[:, 2] = np.sqrt(areas / np.repeat(ratios, len(scales)))
        anchors[:, 3] = anchors[:, 2] * np.repeat(ratios, len(scales))
        anchors[:, 0::2] -= np.tile(anchors[:, 2] * 0.5, (2, 1)).T
        anchors[:, 1::2] -= np.tile(anchors[:, 3] * 0.5, (2, 1)).T
        shape = (image_shape + stride - 1) // stride
        shift_x = (np.arange(0, shape[1]) + 0.5) * stride
        shift_y = (np.arange(0, shape[0]) + 0.5) * stride
        sx, sy = np.meshgrid(shift_x, shift_y)
        shifts = np.vstack((sx.ravel(), sy.ravel(), sx.ravel(), sy.ravel())).transpose()
        A = anchors.shape[0]
        K = shifts.shape[0]
        shifted = (anchors.reshape((1, A, 4)) + shifts.reshape((1, K, 4)).transpose((1, 0, 2))).reshape((K * A, 4))
        all_anchors = np.append(all_anchors, shifted, axis=0)
    return all_anchors.astype(np.float32)


_ANCHORS = _gen_anchors()


def _nms_body(planes, out_hbm,
              sa0, sa1, sa2, sa3, sd0, sd1, se2, se3,
              vx1, vy1, vx2, vy2, var,
              kx1, ky1, kx2, ky2, kar, c0):
    cid = lax.axis_index("c")
    sid = lax.axis_index("s")
    wid = sid * 2 + cid
    i = lax.rem(wid, BATCH)

    for k, dst in enumerate((sa0, sa1, sa2, sa3, sd0, sd1, se2, se3)):
        pltpu.sync_copy(
            planes.at[pl.ds((k * BATCH + i) * CAND_PAD, CAND_PAD)], dst)

    if True:

        def tbody(c, _):
            sl = pl.ds(c * LANES, LANES)
            w = sa2[sl] - sa0[sl]
            h = sa3[sl] - sa1[sl]
            cx = sa0[sl] + 0.5 * w
            cy = sa1[sl] + 0.5 * h
            pcx = cx + (sd0[sl] * 0.1) * w
            pcy = cy + (sd1[sl] * 0.1) * h
            pw = se2[sl] * w
            ph = se3[sl] * h
            x1 = jnp.minimum(jnp.maximum(pcx - 0.5 * pw, 0.0), float(IMG_SIZE))
            y1 = jnp.minimum(jnp.maximum(pcy - 0.5 * ph, 0.0), float(IMG_SIZE))
            x2 = jnp.minimum(jnp.maximum(pcx + 0.5 * pw, 0.0), float(IMG_SIZE))
            y2 = jnp.minimum(jnp.maximum(pcy + 0.5 * ph, 0.0), float(IMG_SIZE))
            vx1[sl] = x1
            vy1[sl] = y1
            vx2[sl] = x2
            vy2[sl] = y2
            var[sl] = (x2 - x1) * (y2 - y1)
            return 0

        lax.fori_loop(0, CAND_PAD // LANES, tbody, 0)

        ifl = lax.convert_element_type(i, jnp.float32)

        def zbody(c, _):
            sl = pl.ds(c * LANES, LANES)
            far = jnp.full((LANES,), 1e9, jnp.float32)
            kx1[sl] = far
            ky1[sl] = far
            kx2[sl] = far
            ky2[sl] = far
            kar[sl] = jnp.zeros((LANES,), jnp.float32)
            c0[sl] = jnp.full((LANES,), ifl, jnp.float32)
            return 0

        lax.fori_loop(0, KEEP_PAD // LANES, zbody, 0)

        lane = lax.iota(jnp.int32, LANES)
        wmask = lane == 0

        def body(j, cnt):
            jv = jnp.full((LANES,), j, jnp.int32)
            bx1 = plsc.load_gather(vx1, [jv])
            by1 = plsc.load_gather(vy1, [jv])
            bx2 = plsc.load_gather(vx2, [jv])
            by2 = plsc.load_gather(vy2, [jv])
            bar = plsc.load_gather(var, [jv])

            def _hits(off):
                sl = pl.ds(off, LANES)
                xx1 = jnp.maximum(kx1[sl], bx1)
                yy1 = jnp.maximum(ky1[sl], by1)
                xx2 = jnp.minimum(kx2[sl], bx2)
                yy2 = jnp.minimum(ky2[sl], by2)
                w = jnp.maximum(xx2 - xx1, 0.0)
                h = jnp.maximum(yy2 - yy1, 0.0)
                inter = w * h
                iou = inter / (kar[sl] + bar - inter + 1e-8)
                return iou > NMS_THRESH

            def ibody(c, supv):
                off = c * (2 * LANES)
                return supv | _hits(off) | _hits(off + LANES)

            active = cnt < POST_NMS_TOPN
            nch = jnp.where(active, (cnt + 2 * LANES - 1) // (2 * LANES),
                            jnp.int32(0))
            supv = lax.fori_loop(0, nch, ibody,
                                 jnp.zeros((LANES,), jnp.bool_))
            sup = jnp.any(supv)

            take = jnp.logical_and(jnp.logical_not(sup), active)
            smask = jnp.logical_and(wmask, jnp.full((LANES,), take))
            cv = jnp.full((LANES,), cnt, jnp.int32)
            plsc.store_scatter(kx1, [cv], bx1, mask=smask)
            plsc.store_scatter(ky1, [cv], by1, mask=smask)
            plsc.store_scatter(kx2, [cv], bx2, mask=smask)
            plsc.store_scatter(ky2, [cv], by2, mask=smask)
            plsc.store_scatter(kar, [cv], bar, mask=smask)

            return cnt + jnp.where(take, jnp.int32(1), jnp.int32(0))

        def bbody(b, cnt):
            nin = jnp.where(cnt < POST_NMS_TOPN, jnp.int32(LANES), jnp.int32(0))

            def cbody(t, c):
                return body(b * LANES + t, c)

            return lax.fori_loop(0, nin, cbody, cnt)

        cnt_f = lax.fori_loop(0, PRE_NMS_TOPN // LANES, bbody, jnp.int32(0))

        def fbody(c, _):
            sl = pl.ds(c * LANES, LANES)
            m = (c * LANES + lane) < cnt_f
            z = jnp.zeros((LANES,), jnp.float32)
            kx1[sl] = jnp.where(m, kx1[sl], z)
            ky1[sl] = jnp.where(m, ky1[sl], z)
            kx2[sl] = jnp.where(m, kx2[sl], z)
            ky2[sl] = jnp.where(m, ky2[sl], z)
            return 0

        lax.fori_loop(0, KEEP_PAD // LANES, fbody, 0)

        base = wid * 5 * KEEP_PAD
        pltpu.sync_copy(c0, out_hbm.at[pl.ds(base, KEEP_PAD)])
        pltpu.sync_copy(kx1, out_hbm.at[pl.ds(base + KEEP_PAD, KEEP_PAD)])
        pltpu.sync_copy(ky1, out_hbm.at[pl.ds(base + 2 * KEEP_PAD, KEEP_PAD)])
        pltpu.sync_copy(kx2, out_hbm.at[pl.ds(base + 3 * KEEP_PAD, KEEP_PAD)])
        pltpu.sync_copy(ky2, out_hbm.at[pl.ds(base + 4 * KEEP_PAD, KEEP_PAD)])


_nms_sc = functools.partial(
    pl.kernel,
    out_type=jax.ShapeDtypeStruct((32 * 5 * KEEP_PAD,), jnp.float32),
    mesh=plsc.VectorSubcoreMesh(core_axis_name="c", subcore_axis_name="s"),
    compiler_params=pltpu.CompilerParams(needs_layout_passes=False),
    scratch_types=(
        [pltpu.VMEM((CAND_PAD,), jnp.float32) for _ in range(8)]
        + [pltpu.VMEM((CAND_PAD,), jnp.float32) for _ in range(5)]
        + [pltpu.VMEM((KEEP_PAD,), jnp.float32) for _ in range(6)]
    ),
)(_nms_body)


NUM_ANCH = _ANCHORS.shape[0]
ANCH_PAD = 49152
N_SHARD = 16
SHARD = ANCH_PAD // N_SHARD
KSHARD = 512


def _topk_sharded(sc):
    scp = jnp.pad(sc, ((0, 0), (0, ANCH_PAD - NUM_ANCH)),
                  constant_values=-1.0)
    sh = scp.reshape(BATCH, N_SHARD, SHARD)
    v1, i1 = lax.top_k(sh, KSHARD)
    gidx = i1 + (jnp.arange(N_SHARD, dtype=jnp.int32) * SHARD)[None, :, None]
    vm = v1.reshape(BATCH, N_SHARD * KSHARD)
    im = gidx.reshape(BATCH, N_SHARD * KSHARD)
    v2, i2 = lax.top_k(vm, PRE_NMS_TOPN)
    idx = jnp.take_along_axis(im, i2, axis=1)
    tau = v2[:, PRE_NMS_TOPN - 1]
    safe = jnp.all(v1[:, :, KSHARD - 1] < tau[:, None])
    return lax.cond(safe, lambda: idx,
                    lambda: lax.top_k(sc, PRE_NMS_TOPN)[1])


def kernel(scores, bbox_deltas, im_info):
    del im_info
    sc = scores[:, :, 0]
    idx = _topk_sharded(sc)

    anchors = jnp.asarray(_ANCHORS)
    anc = anchors[idx]
    dg = jnp.take_along_axis(bbox_deltas, idx[:, :, None], axis=1)
    pad3 = ((0, 0), (0, CAND_PAD - PRE_NMS_TOPN), (0, 0))
    anc = jnp.pad(anc, pad3)
    dg = jnp.pad(dg, pad3)
    planes = jnp.stack(
        [anc[:, :, 0], anc[:, :, 1], anc[:, :, 2], anc[:, :, 3],
         dg[:, :, 0], dg[:, :, 1],
         jnp.exp(dg[:, :, 2] * 0.2), jnp.exp(dg[:, :, 3] * 0.2)],
        axis=0).reshape(-1)

    out = _nms_sc(planes)
    out = out.reshape(32, 5, KEEP_PAD)[:BATCH]
    return jnp.transpose(out, (0, 2, 1))[:, :POST_NMS_TOPN, :]
